# baseline (speedup 1.0000x reference)
import functools
import math

import jax
import jax.numpy as jnp
from jax import lax
from jax.experimental import pallas as pl
from jax.experimental.pallas import tpu as pltpu
from jax.experimental.pallas import tpu_sc as plsc

_D = 1024
_LANES = 16
_NC = 2
_NS = 16
_NW = _NC * _NS


@functools.lru_cache(maxsize=None)
def _make_emb(n_tok: int, d: int):
    per_w = n_tok // _NW
    chunk = 64
    n_chunks = per_w // chunk
    scale = math.sqrt(d)
    mesh = plsc.VectorSubcoreMesh(core_axis_name="c", subcore_axis_name="s")

    @functools.partial(
        pl.kernel,
        out_type=jax.ShapeDtypeStruct((n_tok, d), jnp.float32),
        mesh=mesh,
        scratch_types=[
            pltpu.VMEM((per_w,), jnp.int32),
            pltpu.VMEM((chunk, d), jnp.float32),
            pltpu.SemaphoreType.DMA,
        ],
    )
    def emb(x_hbm, table_hbm, out_hbm, idx_v, rows_v, sem):
        wid = lax.axis_index("s") * _NC + lax.axis_index("c")
        base = wid * per_w
        pltpu.sync_copy(x_hbm.at[pl.ds(base, per_w)], idx_v)

        def do_chunk(g, carry):
            off = g * chunk
            pltpu.async_copy(
                table_hbm.at[idx_v.at[pl.ds(off, chunk)]], rows_v, sem
            ).wait()

            def scale_row(r, c2):
                def scale_vec(j, c3):
                    sl = pl.ds(j * _LANES, _LANES)
                    rows_v[r, sl] = rows_v[r, sl] * scale
                    return c3
                return lax.fori_loop(0, d // _LANES, scale_vec, c2)

            lax.fori_loop(0, chunk, scale_row, 0)
            pltpu.sync_copy(rows_v, out_hbm.at[pl.ds(base + off, chunk)])
            return carry

        lax.fori_loop(0, n_chunks, do_chunk, 0)

    return emb


def kernel(x, table):
    b, s = x.shape
    n_tok = b * s
    d = table.shape[1]
    out = _make_emb(n_tok, d)(x.reshape(n_tok).astype(jnp.int32), table)
    return out.reshape(b, s, d)


# speedup vs baseline: 2.8497x; 2.8497x over previous
import functools
import math

import jax
import jax.numpy as jnp
from jax import lax
from jax.experimental import pallas as pl
from jax.experimental.pallas import tpu as pltpu
from jax.experimental.pallas import tpu_sc as plsc

_LANES = 16
_NC = 2
_NS = 16
_NW = _NC * _NS


@functools.lru_cache(maxsize=None)
def _make_emb(n_tok: int, d: int):
    per_w = n_tok // _NW
    chunk = 32
    n_chunks = per_w // chunk
    vecs_per_row = d // _LANES
    scale = math.sqrt(d)
    mesh = plsc.VectorSubcoreMesh(core_axis_name="c", subcore_axis_name="s")

    @functools.partial(
        pl.kernel,
        out_type=jax.ShapeDtypeStruct((n_tok, d), jnp.float32),
        mesh=mesh,
        scratch_types=[
            pltpu.VMEM((per_w,), jnp.int32),
            pltpu.VMEM((chunk, d), jnp.float32),
            pltpu.VMEM((chunk, d), jnp.float32),
            pltpu.SemaphoreType.DMA,
            pltpu.SemaphoreType.DMA,
            pltpu.SemaphoreType.DMA,
            pltpu.SemaphoreType.DMA,
        ],
    )
    def emb(x_hbm, table_hbm, out_hbm, idx_v, rows_a, rows_b,
            gsem_a, gsem_b, ssem_a, ssem_b):
        wid = lax.axis_index("s") * _NC + lax.axis_index("c")
        base = wid * per_w
        pltpu.sync_copy(x_hbm.at[pl.ds(base, per_w)], idx_v)

        bufs = (rows_a, rows_b)
        gsems = (gsem_a, gsem_b)
        ssems = (ssem_a, ssem_b)

        def gather(g):
            p = g % 2
            return pltpu.async_copy(
                table_hbm.at[idx_v.at[pl.ds(g * chunk, chunk)]],
                bufs[p], gsems[p])

        def scale_chunk(buf):
            def scale_row(r, c):
                for j in range(vecs_per_row):
                    sl = pl.ds(j * _LANES, _LANES)
                    buf[r, sl] = buf[r, sl] * scale
                return c
            lax.fori_loop(0, chunk, scale_row, 0)

        store_h = [None] * n_chunks
        gather_h = [None] * n_chunks
        gather_h[0] = gather(0)
        for g in range(n_chunks):
            p = g % 2
            if g + 1 < n_chunks:
                if g >= 1:
                    store_h[g - 1].wait()
                gather_h[g + 1] = gather(g + 1)
            gather_h[g].wait()
            scale_chunk(bufs[p])
            store_h[g] = pltpu.async_copy(
                bufs[p], out_hbm.at[pl.ds(base + g * chunk, chunk)], ssems[p])
        store_h[n_chunks - 2].wait()
        store_h[n_chunks - 1].wait()

    return emb


def kernel(x, table):
    b, s = x.shape
    n_tok = b * s
    d = table.shape[1]
    out = _make_emb(n_tok, d)(x.reshape(n_tok).astype(jnp.int32), table)
    return out.reshape(b, s, d)
